# baseline (speedup 1.0000x reference)
import functools

import jax
import jax.numpy as jnp
from jax import lax
from jax.experimental import pallas as pl
from jax.experimental.pallas import tpu as pltpu
from jax.experimental.pallas import tpu_sc as plsc

F = 100
V = 100
D = 128
B = 4096
LOGB = 12

R = B * F
NW = 32
RW = R // NW
CHUNK = 64
NCHUNK = RW // CHUNK
LANES = 16
NBUF = 4
LAG = 3


def _gather_body(xt_hbm, tbl_hbm, out_hbm, idxbuf, rows, shared_tbl, *sems):
  gsems = sems[:NBUF]
  ssems = sems[NBUF:]
  sid = lax.axis_index("s")
  wid = lax.axis_index("c") * 16 + sid
  base = wid * RW

  TSLICE = 624
  toff = sid * TSLICE
  tlen = TSLICE + (F * V - 16 * TSLICE)
  @pl.when(sid < 15)
  def _stage_table():
    pltpu.sync_copy(tbl_hbm.at[pl.ds(toff, TSLICE)],
                    shared_tbl.at[pl.ds(toff, TSLICE)])
  @pl.when(sid == 15)
  def _stage_tail():
    pltpu.sync_copy(tbl_hbm.at[pl.ds(15 * TSLICE, tlen)],
                    shared_tbl.at[pl.ds(15 * TSLICE, tlen)])

  pltpu.sync_copy(xt_hbm.at[pl.ds(base, RW)], idxbuf)

  iota = lax.iota(jnp.int32, LANES)

  def idx_step(j, _):
    xv = idxbuf[pl.ds(j * LANES, LANES)]
    pos = base + j * LANES + iota
    fv = lax.shift_right_logical(pos, LOGB)
    idxbuf[pl.ds(j * LANES, LANES)] = fv * V + xv
    return 0

  lax.fori_loop(0, RW // LANES, idx_step, 0)
  plsc.subcore_barrier()

  def gfire(c):
    b = c % NBUF
    return pltpu.async_copy(
        shared_tbl.at[idxbuf.at[pl.ds(c * CHUNK, CHUNK)]], rows.at[b],
        gsems[b])

  def sfire(c):
    b = c % NBUF
    return pltpu.async_copy(
        rows.at[b], out_hbm.at[pl.ds(base + c * CHUNK, CHUNK)], ssems[b])

  gdesc = {}
  sdesc = {}
  for c in range(NBUF):
    gdesc[c] = gfire(c)
  for c in range(NCHUNK):
    gdesc.pop(c).wait()
    sdesc[c] = sfire(c)
    c2 = c - LAG
    if c2 >= 0 and c2 + NBUF < NCHUNK:
      sdesc.pop(c2).wait()
      gdesc[c2 + NBUF] = gfire(c2 + NBUF)
  for c in sorted(sdesc):
    sdesc.pop(c).wait()


@jax.jit
def _run(xt_flat, tbl_flat):
  mesh = plsc.VectorSubcoreMesh(core_axis_name="c", subcore_axis_name="s")
  k = pl.kernel(
      _gather_body,
      out_type=jax.ShapeDtypeStruct((R, D), jnp.float32),
      mesh=mesh,
      scratch_types=[
          pltpu.VMEM((RW,), jnp.int32),
          pltpu.VMEM((NBUF, CHUNK, D), jnp.float32),
          pltpu.VMEM_SHARED((F * V, D), jnp.float32),
      ] + [pltpu.SemaphoreType.DMA] * (2 * NBUF),
  )
  return k(xt_flat, tbl_flat)


def kernel(x, tables):
  xt_flat = x.T.reshape(R)
  tbl_flat = tables.reshape(F * V, D)
  out = _run(xt_flat, tbl_flat)
  return out.reshape(F, B, D).transpose(1, 0, 2)


# speedup vs baseline: 1.0850x; 1.0850x over previous
import functools

import jax
import jax.numpy as jnp
from jax import lax
from jax.experimental import pallas as pl
from jax.experimental.pallas import tpu as pltpu
from jax.experimental.pallas import tpu_sc as plsc

F = 100
V = 100
D = 128
B = 4096
LOGB = 12

R = B * F
NW = 32
RW = R // NW
CHUNK = 64
NCHUNK = RW // CHUNK
LANES = 16
NBUF = 4
LAG = 2


def _gather_body(xt_hbm, tbl_hbm, out_hbm, idxbuf, rows, shared_tbl, *sems):
  gsems = sems[:NBUF]
  ssems = sems[NBUF:]
  sid = lax.axis_index("s")
  wid = lax.axis_index("c") * 16 + sid
  base = wid * RW

  TSLICE = 624
  toff = sid * TSLICE
  tlen = TSLICE + (F * V - 16 * TSLICE)
  @pl.when(sid < 15)
  def _stage_table():
    pltpu.sync_copy(tbl_hbm.at[pl.ds(toff, TSLICE)],
                    shared_tbl.at[pl.ds(toff, TSLICE)])
  @pl.when(sid == 15)
  def _stage_tail():
    pltpu.sync_copy(tbl_hbm.at[pl.ds(15 * TSLICE, tlen)],
                    shared_tbl.at[pl.ds(15 * TSLICE, tlen)])

  pltpu.sync_copy(xt_hbm.at[pl.ds(base, RW)], idxbuf)

  iota = lax.iota(jnp.int32, LANES)

  def idx_step(j, _):
    xv = idxbuf[pl.ds(j * LANES, LANES)]
    pos = base + j * LANES + iota
    fv = lax.shift_right_logical(pos, LOGB)
    idxbuf[pl.ds(j * LANES, LANES)] = fv * V + xv
    return 0

  lax.fori_loop(0, RW // LANES, idx_step, 0)
  plsc.subcore_barrier()

  def gfire(c):
    b = c % NBUF
    return pltpu.async_copy(
        shared_tbl.at[idxbuf.at[pl.ds(c * CHUNK, CHUNK)]], rows.at[b],
        gsems[b])

  def sfire(c):
    b = c % NBUF
    return pltpu.async_copy(
        rows.at[b], out_hbm.at[pl.ds(base + c * CHUNK, CHUNK)], ssems[b])

  gdesc = {}
  sdesc = {}
  for c in range(NBUF):
    gdesc[c] = gfire(c)
  for c in range(NCHUNK):
    gdesc.pop(c).wait()
    sdesc[c] = sfire(c)
    c2 = c - LAG
    if c2 >= 0 and c2 + NBUF < NCHUNK:
      sdesc.pop(c2).wait()
      gdesc[c2 + NBUF] = gfire(c2 + NBUF)
  for c in sorted(sdesc):
    sdesc.pop(c).wait()


@jax.jit
def _run(xt_flat, tbl_flat):
  mesh = plsc.VectorSubcoreMesh(core_axis_name="c", subcore_axis_name="s")
  k = pl.kernel(
      _gather_body,
      out_type=jax.ShapeDtypeStruct((R, D), jnp.float32),
      mesh=mesh,
      scratch_types=[
          pltpu.VMEM((RW,), jnp.int32),
          pltpu.VMEM((NBUF, CHUNK, D), jnp.float32),
          pltpu.VMEM_SHARED((F * V, D), jnp.float32),
      ] + [pltpu.SemaphoreType.DMA] * (2 * NBUF),
  )
  return k(xt_flat, tbl_flat)


def kernel(x, tables):
  xt_flat = x.T.reshape(R)
  tbl_flat = tables.reshape(F * V, D)
  out = _run(xt_flat, tbl_flat)
  return out.reshape(F, B, D).transpose(1, 0, 2)
